# Initial kernel scaffold; baseline (speedup 1.0000x reference)
import jax
import jax.numpy as jnp
from jax.experimental import pallas as pl


def kernel(x, emb_word, emb_bigram, emb_trigram, fc1_w, fc1_b, fc2_w, fc2_b):
    raise NotImplementedError("write your pallas kernel here")


# speedup vs baseline: 2.1205x; 2.1205x over previous
import functools

import jax
import jax.numpy as jnp
from jax import lax
from jax.experimental import pallas as pl
from jax.experimental.pallas import tpu as pltpu
from jax.experimental.pallas import tpu_sc as plsc

VOCAB = 100000
EMBED = 64
HIDDEN = 256
NUM_CLASSES = 10
PAD = VOCAB - 1
BATCH = 1024
SEQ = 200
N = BATCH * SEQ

NW = 32
PER_W = N // NW
CHUNK = 128
N_CHUNKS = PER_W // CHUNK

BT = 512
NB = N // BT


def _sc_gather(idx_w, idx_b, idx_t, emb_word, emb_bigram, emb_trigram):
    mesh = plsc.VectorSubcoreMesh(core_axis_name="c", subcore_axis_name="s")
    row_ty = jax.ShapeDtypeStruct((N, EMBED), jnp.float32)

    @functools.partial(
        pl.kernel,
        mesh=mesh,
        out_type=(row_ty, row_ty, row_ty),
        compiler_params=pltpu.CompilerParams(use_tc_tiling_on_sc=False),
        scratch_types=[
            pltpu.VMEM((CHUNK,), jnp.int32),
            pltpu.VMEM((CHUNK, EMBED), jnp.float32),
            pltpu.SemaphoreType.DMA,
        ],
    )
    def gather_kernel(iw, ib, it, tw, tb, tt, ow, ob, ot, idx_v, rows_v, sem):
        info = plsc.get_sparse_core_info()
        nc = info.num_cores
        wid = lax.axis_index("s") * nc + lax.axis_index("c")
        base = wid * PER_W
        for s, (ixs, tbl, out) in enumerate(
                ((iw, tw, ow), (ib, tb, ob), (it, tt, ot))):
            def body(c, _, s=s, ixs=ixs, tbl=tbl, out=out):
                tb0 = base + c * CHUNK
                pltpu.sync_copy(ixs.at[pl.ds(tb0, CHUNK)], idx_v)
                pltpu.async_copy(tbl.at[idx_v], rows_v, sem).wait()
                pltpu.sync_copy(rows_v, out.at[pl.ds(tb0, CHUNK), :])
                return 0
            lax.fori_loop(0, N_CHUNKS, body, 0)

    return gather_kernel(idx_w, idx_b, idx_t, emb_word, emb_bigram, emb_trigram)


def _mlp_kernel(gw_ref, gb_ref, gt_ref, idx3_ref, w1a_ref, w1b_ref, w1c_ref,
                b1_ref, w2_ref, b2_ref, o_ref):
    idx3 = idx3_ref[0]
    h = b1_ref[...].astype(jnp.float32)
    for s, (g_ref, w_ref) in enumerate(
            ((gw_ref, w1a_ref), (gb_ref, w1b_ref), (gt_ref, w1c_ref))):
        m = (idx3[:, s:s + 1] != PAD).astype(jnp.float32)
        xm = (g_ref[...] * m).astype(jnp.bfloat16)
        h = h + jnp.dot(xm, w_ref[...], preferred_element_type=jnp.float32)
    h = jnp.maximum(h, 0.0).astype(jnp.bfloat16)
    o = jnp.dot(h, w2_ref[...], preferred_element_type=jnp.float32)
    o_ref[...] = o + b2_ref[...]


def _tc_mlp(gw, gb, gt, idx3, fc1_w, fc1_b, fc2_w, fc2_b):
    w1 = fc1_w.T.astype(jnp.bfloat16)
    w1a, w1b, w1c = w1[:EMBED], w1[EMBED:2 * EMBED], w1[2 * EMBED:]
    w2 = fc2_w.T.astype(jnp.bfloat16)
    b1 = fc1_b.reshape(1, HIDDEN)
    b2 = fc2_b.reshape(1, NUM_CLASSES)
    g_spec = pl.BlockSpec((BT, EMBED), lambda i: (i, 0))
    w_spec = pl.BlockSpec((EMBED, HIDDEN), lambda i: (0, 0))
    return pl.pallas_call(
        _mlp_kernel,
        grid=(NB,),
        in_specs=[
            g_spec, g_spec, g_spec,
            pl.BlockSpec((1, BT, 3), lambda i: (i, 0, 0)),
            w_spec, w_spec, w_spec,
            pl.BlockSpec((1, HIDDEN), lambda i: (0, 0)),
            pl.BlockSpec((HIDDEN, NUM_CLASSES), lambda i: (0, 0)),
            pl.BlockSpec((1, NUM_CLASSES), lambda i: (0, 0)),
        ],
        out_specs=pl.BlockSpec((BT, NUM_CLASSES), lambda i: (i, 0)),
        out_shape=jax.ShapeDtypeStruct((N, NUM_CLASSES), jnp.float32),
    )(gw, gb, gt, idx3, w1a, w1b, w1c, b1, w2, b2)


def kernel(x, emb_word, emb_bigram, emb_trigram, fc1_w, fc1_b, fc2_w, fc2_b):
    idx = x.reshape(3, N)
    idx3 = jnp.transpose(idx, (1, 0)).reshape(NB, BT, 3)
    gw, gb, gt = _sc_gather(idx[0], idx[1], idx[2],
                            emb_word, emb_bigram, emb_trigram)
    out = _tc_mlp(gw, gb, gt, idx3, fc1_w, fc1_b, fc2_w, fc2_b)
    return out.reshape(BATCH, SEQ, NUM_CLASSES)


# speedup vs baseline: 2.6543x; 1.2517x over previous
import functools

import jax
import jax.numpy as jnp
import numpy as np
from jax import lax
from jax.experimental import pallas as pl
from jax.experimental.pallas import tpu as pltpu
from jax.experimental.pallas import tpu_sc as plsc

VOCAB = 100000
EMBED = 64
HIDDEN = 256
NUM_CLASSES = 10
PAD = VOCAB - 1
BATCH = 1024
SEQ = 200
N = BATCH * SEQ

NW = 32
PER_W = N // NW
CHUNK = 128
N_CHUNKS = PER_W // CHUNK

BT = 512
NB = N // BT
BP = BT // 2

_SEL2 = np.zeros((2, 2 * EMBED), dtype=np.float32)
_SEL2[0, :EMBED] = 1.0
_SEL2[1, EMBED:] = 1.0


def _sc_gather(idx_w, idx_b, idx_t, emb_word, emb_bigram, emb_trigram):
    mesh = plsc.VectorSubcoreMesh(core_axis_name="c", subcore_axis_name="s")
    row_ty = jax.ShapeDtypeStruct((N, EMBED), jnp.float32)

    @functools.partial(
        pl.kernel,
        mesh=mesh,
        out_type=(row_ty, row_ty, row_ty),
        compiler_params=pltpu.CompilerParams(use_tc_tiling_on_sc=False),
        scratch_types=[
            pltpu.VMEM((CHUNK,), jnp.int32),
            pltpu.VMEM((CHUNK,), jnp.int32),
            pltpu.VMEM((CHUNK, EMBED), jnp.float32),
            pltpu.VMEM((CHUNK, EMBED), jnp.float32),
            pltpu.SemaphoreType.DMA,
            pltpu.SemaphoreType.DMA,
        ],
    )
    def gather_kernel(iw, ib, it, tw, tb, tt, ow, ob, ot,
                      idx0, idx1, rows0, rows1, sem0, sem1):
        info = plsc.get_sparse_core_info()
        nc = info.num_cores
        wid = lax.axis_index("s") * nc + lax.axis_index("c")
        base = wid * PER_W
        idx_bufs = (idx0, idx1)
        row_bufs = (rows0, rows1)
        sems = (sem0, sem1)

        for s, (ixs, tbl, out) in enumerate(
                ((iw, tw, ow), (ib, tb, ob), (it, tt, ot))):
            for b in range(2):
                pltpu.sync_copy(ixs.at[pl.ds(base + b * CHUNK, CHUNK)],
                                idx_bufs[b])
                pltpu.async_copy(tbl.at[idx_bufs[b]], row_bufs[b], sems[b])

            def pair(jj, _, ixs=ixs, tbl=tbl, out=out):
                for b in range(2):
                    j = jj * 2 + b
                    pltpu.make_async_copy(tbl.at[idx_bufs[b]], row_bufs[b],
                                          sems[b]).wait()
                    pltpu.sync_copy(row_bufs[b],
                                    out.at[pl.ds(base + j * CHUNK, CHUNK), :])

                    @pl.when(j + 2 < N_CHUNKS)
                    def _():
                        pltpu.sync_copy(
                            ixs.at[pl.ds(base + (j + 2) * CHUNK, CHUNK)],
                            idx_bufs[b])
                        pltpu.async_copy(tbl.at[idx_bufs[b]], row_bufs[b],
                                         sems[b])
                return 0

            lax.fori_loop(0, N_CHUNKS // 2, pair, 0)

    return gather_kernel(idx_w, idx_b, idx_t, emb_word, emb_bigram, emb_trigram)


def _mlp_kernel(gw_ref, gb_ref, gt_ref, iw_ref, ib_ref, it_ref, sel2_ref,
                w1a_ref, w1b_ref, w1c_ref, b1_ref, w2_ref, b2_ref, o_ref):
    h = b1_ref[...].astype(jnp.float32)
    for g_ref, i_ref, w_ref in ((gw_ref, iw_ref, w1a_ref),
                                (gb_ref, ib_ref, w1b_ref),
                                (gt_ref, it_ref, w1c_ref)):
        m2 = (i_ref[0] != PAD).astype(jnp.float32)
        mask = jnp.dot(m2, sel2_ref[...],
                       preferred_element_type=jnp.float32)
        xm = (g_ref[...] * mask).astype(jnp.bfloat16)
        h = h + jnp.dot(xm, w_ref[...], preferred_element_type=jnp.float32)
    h = jnp.maximum(h, 0.0).astype(jnp.bfloat16)
    o = jnp.dot(h, w2_ref[...], preferred_element_type=jnp.float32)
    o_ref[...] = o + b2_ref[...]


def _tc_mlp(gw, gb, gt, iw, ib, it, fc1_w, fc1_b, fc2_w, fc2_b):
    w1 = fc1_w.T.astype(jnp.bfloat16)
    z = jnp.zeros((EMBED, HIDDEN), dtype=jnp.bfloat16)
    wd = []
    for s in range(3):
        ws = w1[s * EMBED:(s + 1) * EMBED]
        wd.append(jnp.block([[ws, z], [z, ws]]))
    w2 = fc2_w.T.astype(jnp.bfloat16)
    z2 = jnp.zeros((HIDDEN, NUM_CLASSES), dtype=jnp.bfloat16)
    w2d = jnp.block([[w2, z2], [z2, w2]])
    b1d = jnp.concatenate([fc1_b, fc1_b]).reshape(1, 2 * HIDDEN)
    b2d = jnp.concatenate([fc2_b, fc2_b]).reshape(1, 2 * NUM_CLASSES)
    sel2 = jnp.asarray(_SEL2)

    g_spec = pl.BlockSpec((BP, 2 * EMBED), lambda i: (i, 0))
    i_spec = pl.BlockSpec((1, BP, 2), lambda i: (i, 0, 0))
    w_spec = pl.BlockSpec((2 * EMBED, 2 * HIDDEN), lambda i: (0, 0))
    return pl.pallas_call(
        _mlp_kernel,
        grid=(NB,),
        in_specs=[
            g_spec, g_spec, g_spec,
            i_spec, i_spec, i_spec,
            pl.BlockSpec((2, 2 * EMBED), lambda i: (0, 0)),
            w_spec, w_spec, w_spec,
            pl.BlockSpec((1, 2 * HIDDEN), lambda i: (0, 0)),
            pl.BlockSpec((2 * HIDDEN, 2 * NUM_CLASSES), lambda i: (0, 0)),
            pl.BlockSpec((1, 2 * NUM_CLASSES), lambda i: (0, 0)),
        ],
        out_specs=pl.BlockSpec((BP, 2 * NUM_CLASSES), lambda i: (i, 0)),
        out_shape=jax.ShapeDtypeStruct((N // 2, 2 * NUM_CLASSES), jnp.float32),
    )(gw, gb, gt, iw, ib, it, sel2, *wd, b1d, w2d, b2d)


def kernel(x, emb_word, emb_bigram, emb_trigram, fc1_w, fc1_b, fc2_w, fc2_b):
    idx = x.reshape(3, N)
    gw, gb, gt = _sc_gather(idx[0], idx[1], idx[2],
                            emb_word, emb_bigram, emb_trigram)
    gw2 = gw.reshape(N // 2, 2 * EMBED)
    gb2 = gb.reshape(N // 2, 2 * EMBED)
    gt2 = gt.reshape(N // 2, 2 * EMBED)
    iw = idx[0].reshape(NB, BP, 2)
    ib = idx[1].reshape(NB, BP, 2)
    it = idx[2].reshape(NB, BP, 2)
    out = _tc_mlp(gw2, gb2, gt2, iw, ib, it, fc1_w, fc1_b, fc2_w, fc2_b)
    return out.reshape(BATCH, SEQ, NUM_CLASSES)


# speedup vs baseline: 3.6965x; 1.3926x over previous
import functools

import jax
import jax.numpy as jnp
from jax import lax
from jax.experimental import pallas as pl
from jax.experimental.pallas import tpu as pltpu
from jax.experimental.pallas import tpu_sc as plsc

VOCAB = 100000
EMBED = 64
HIDDEN = 256
NUM_CLASSES = 10
PAD = VOCAB - 1
BATCH = 1024
SEQ = 200
N = BATCH * SEQ

NW = 32
PER_W = N // NW
CHUNK = 128
N_CHUNKS = PER_W // CHUNK

BT = 1024
NB = N // BT
BP = BT // 2


def _sc_gather(idx_w, idx_b, idx_t, emb_word, emb_bigram, emb_trigram):
    mesh = plsc.VectorSubcoreMesh(core_axis_name="c", subcore_axis_name="s")
    row_ty = jax.ShapeDtypeStruct((N, EMBED), jnp.float32)

    @functools.partial(
        pl.kernel,
        mesh=mesh,
        out_type=(row_ty, row_ty, row_ty),
        compiler_params=pltpu.CompilerParams(
            use_tc_tiling_on_sc=False, needs_layout_passes=False),
        scratch_types=[
            pltpu.VMEM((CHUNK,), jnp.int32),
            pltpu.VMEM((CHUNK,), jnp.int32),
            pltpu.VMEM((CHUNK, EMBED), jnp.float32),
            pltpu.VMEM((CHUNK, EMBED), jnp.float32),
            pltpu.SemaphoreType.DMA,
            pltpu.SemaphoreType.DMA,
        ],
    )
    def gather_kernel(iw, ib, it, tw, tb, tt, ow, ob, ot,
                      idx0, idx1, rows0, rows1, sem0, sem1):
        info = plsc.get_sparse_core_info()
        nc = info.num_cores
        wid = lax.axis_index("s") * nc + lax.axis_index("c")
        base = wid * PER_W
        idx_bufs = (idx0, idx1)
        row_bufs = (rows0, rows1)
        sems = (sem0, sem1)
        lane = lax.iota(jnp.int32, 16)

        def fixup(idxb, rowsb):
            def group(g, _):
                iv = idxb[pl.ds(g * 16, 16)]
                hit = (iv == PAD)
                any_hit = lax.reduce_max(hit.astype(jnp.int32), axes=(0,))

                @pl.when(any_hit > 0)
                def _():
                    rowv = g * 16 + lane

                    def word(w, _):
                        colv = jnp.zeros((16,), jnp.int32) + w
                        plsc.store_scatter(rowsb, [rowv, colv],
                                           jnp.zeros((16,), jnp.float32),
                                           mask=hit)
                        return 0
                    lax.fori_loop(0, EMBED, word, 0)
                return 0
            lax.fori_loop(0, CHUNK // 16, group, 0)

        for s, (ixs, tbl, out) in enumerate(
                ((iw, tw, ow), (ib, tb, ob), (it, tt, ot))):
            for b in range(2):
                pltpu.sync_copy(ixs.at[pl.ds(base + b * CHUNK, CHUNK)],
                                idx_bufs[b])
                pltpu.async_copy(tbl.at[idx_bufs[b]], row_bufs[b], sems[b])

            def pair(jj, _, ixs=ixs, tbl=tbl, out=out):
                for b in range(2):
                    j = jj * 2 + b
                    pltpu.make_async_copy(tbl.at[idx_bufs[b]], row_bufs[b],
                                          sems[b]).wait()
                    fixup(idx_bufs[b], row_bufs[b])
                    pltpu.sync_copy(row_bufs[b],
                                    out.at[pl.ds(base + j * CHUNK, CHUNK), :])

                    @pl.when(j + 2 < N_CHUNKS)
                    def _():
                        pltpu.sync_copy(
                            ixs.at[pl.ds(base + (j + 2) * CHUNK, CHUNK)],
                            idx_bufs[b])
                        pltpu.async_copy(tbl.at[idx_bufs[b]], row_bufs[b],
                                         sems[b])
                return 0

            lax.fori_loop(0, N_CHUNKS // 2, pair, 0)

    return gather_kernel(idx_w, idx_b, idx_t, emb_word, emb_bigram, emb_trigram)


def _mlp_kernel(gw_ref, gb_ref, gt_ref, w1_ref, b1_ref, w2_ref, b2_ref,
                ee_ref, eo_ref, o_ref):
    xcat = jnp.concatenate(
        [gw_ref[...], gb_ref[...], gt_ref[...]], axis=1
    ).astype(jnp.bfloat16)
    h = jnp.dot(xcat, w1_ref[...], preferred_element_type=jnp.float32)
    h = jnp.maximum(h + b1_ref[...], 0.0).astype(jnp.bfloat16)
    o = jnp.dot(h, w2_ref[...], preferred_element_type=jnp.float32)
    o = o + b2_ref[...]
    ot = jnp.transpose(o, (1, 0))
    ev = jnp.dot(ot[0:NUM_CLASSES, :], ee_ref[...],
                 preferred_element_type=jnp.float32)
    od = jnp.dot(ot[16:16 + NUM_CLASSES, :], eo_ref[...],
                 preferred_element_type=jnp.float32)
    o_ref[...] = ev + od


def _tc_mlp(gw, gb, gt, fc1_w, fc1_b, fc2_w, fc2_b):
    w1 = fc1_w.T.astype(jnp.bfloat16)
    z = jnp.zeros((EMBED, HIDDEN), dtype=jnp.bfloat16)
    wd = []
    for s in range(3):
        ws = w1[s * EMBED:(s + 1) * EMBED]
        wd.append(jnp.block([[ws, z], [z, ws]]))
    wcat = jnp.concatenate(wd, axis=0)
    w2 = fc2_w.T.astype(jnp.bfloat16)
    w2p = jnp.zeros((2 * HIDDEN, 32), dtype=jnp.bfloat16)
    w2p = w2p.at[:HIDDEN, :NUM_CLASSES].set(w2)
    w2p = w2p.at[HIDDEN:, 16:16 + NUM_CLASSES].set(w2)
    b1d = jnp.concatenate([fc1_b, fc1_b]).reshape(1, 2 * HIDDEN)
    b2p = jnp.zeros((1, 32), dtype=jnp.float32)
    b2p = b2p.at[0, :NUM_CLASSES].set(fc2_b)
    b2p = b2p.at[0, 16:16 + NUM_CLASSES].set(fc2_b)

    pr = lax.iota(jnp.int32, BP).reshape(BP, 1)
    qc = lax.iota(jnp.int32, BT).reshape(1, BT)
    ee = (qc == 2 * pr).astype(jnp.float32)
    eo = (qc == 2 * pr + 1).astype(jnp.float32)

    g_spec = pl.BlockSpec((BP, 2 * EMBED), lambda i: (i, 0))
    return pl.pallas_call(
        _mlp_kernel,
        grid=(NB,),
        in_specs=[
            g_spec, g_spec, g_spec,
            pl.BlockSpec((3 * 2 * EMBED, 2 * HIDDEN), lambda i: (0, 0)),
            pl.BlockSpec((1, 2 * HIDDEN), lambda i: (0, 0)),
            pl.BlockSpec((2 * HIDDEN, 32), lambda i: (0, 0)),
            pl.BlockSpec((1, 32), lambda i: (0, 0)),
            pl.BlockSpec((BP, BT), lambda i: (0, 0)),
            pl.BlockSpec((BP, BT), lambda i: (0, 0)),
        ],
        out_specs=pl.BlockSpec((NUM_CLASSES, BT), lambda i: (0, i)),
        out_shape=jax.ShapeDtypeStruct((NUM_CLASSES, N), jnp.float32),
    )(gw, gb, gt, wcat, b1d, w2p, b2p, ee, eo)


def kernel(x, emb_word, emb_bigram, emb_trigram, fc1_w, fc1_b, fc2_w, fc2_b):
    idx = jnp.transpose(x, (0, 2, 1)).reshape(3, N)
    gw, gb, gt = _sc_gather(idx[0], idx[1], idx[2],
                            emb_word, emb_bigram, emb_trigram)
    gw2 = gw.reshape(N // 2, 2 * EMBED)
    gb2 = gb.reshape(N // 2, 2 * EMBED)
    gt2 = gt.reshape(N // 2, 2 * EMBED)
    o_np = _tc_mlp(gw2, gb2, gt2, fc1_w, fc1_b, fc2_w, fc2_b)
    return o_np.reshape(NUM_CLASSES, SEQ, BATCH).transpose(2, 1, 0)


# speedup vs baseline: 4.1186x; 1.1142x over previous
import functools

import jax
import jax.numpy as jnp
from jax import lax
from jax.experimental import pallas as pl
from jax.experimental.pallas import tpu as pltpu
from jax.experimental.pallas import tpu_sc as plsc

VOCAB = 100000
EMBED = 64
HIDDEN = 256
NUM_CLASSES = 10
PAD = VOCAB - 1
BATCH = 1024
SEQ = 200
N = BATCH * SEQ

NW = 32
CHUNK = 128
NSPLIT = 2
NT = N // NSPLIT

BT = 1024
BP = BT // 2


def _sc_gather(idx_w, idx_b, idx_t, emb_word, emb_bigram, emb_trigram, tok0):
    mesh = plsc.VectorSubcoreMesh(core_axis_name="c", subcore_axis_name="s")
    row_ty = jax.ShapeDtypeStruct((NT, EMBED), jnp.float32)
    per_w = NT // NW
    n_chunks = per_w // CHUNK
    assert per_w % CHUNK == 0

    @functools.partial(
        pl.kernel,
        mesh=mesh,
        out_type=(row_ty, row_ty, row_ty),
        compiler_params=pltpu.CompilerParams(
            use_tc_tiling_on_sc=False, needs_layout_passes=False),
        scratch_types=[
            pltpu.VMEM((CHUNK,), jnp.int32),
            pltpu.VMEM((CHUNK,), jnp.int32),
            pltpu.VMEM((CHUNK, EMBED), jnp.float32),
            pltpu.VMEM((CHUNK, EMBED), jnp.float32),
            pltpu.SemaphoreType.DMA,
            pltpu.SemaphoreType.DMA,
        ],
    )
    def gather_kernel(iw, ib, it, tw, tb, tt, ow, ob, ot,
                      idx0, idx1, rows0, rows1, sem0, sem1):
        info = plsc.get_sparse_core_info()
        nc = info.num_cores
        wid = lax.axis_index("s") * nc + lax.axis_index("c")
        src_base = tok0 + wid * per_w
        dst_base = wid * per_w
        idx_bufs = (idx0, idx1)
        row_bufs = (rows0, rows1)
        sems = (sem0, sem1)
        lane = lax.iota(jnp.int32, 16)

        def fixup(idxb, rowsb):
            def group(g, _):
                iv = idxb[pl.ds(g * 16, 16)]
                hit = (iv == PAD)
                any_hit = lax.reduce_max(hit.astype(jnp.int32), axes=(0,))

                @pl.when(any_hit > 0)
                def _():
                    rowv = g * 16 + lane

                    def word(w, _):
                        colv = jnp.zeros((16,), jnp.int32) + w
                        plsc.store_scatter(rowsb, [rowv, colv],
                                           jnp.zeros((16,), jnp.float32),
                                           mask=hit)
                        return 0
                    lax.fori_loop(0, EMBED, word, 0)
                return 0
            lax.fori_loop(0, CHUNK // 16, group, 0)

        for s, (ixs, tbl, out) in enumerate(
                ((iw, tw, ow), (ib, tb, ob), (it, tt, ot))):
            for b in range(2):
                pltpu.sync_copy(ixs.at[pl.ds(src_base + b * CHUNK, CHUNK)],
                                idx_bufs[b])
                pltpu.async_copy(tbl.at[idx_bufs[b]], row_bufs[b], sems[b])

            def pair(jj, _, ixs=ixs, tbl=tbl, out=out):
                for b in range(2):
                    j = jj * 2 + b
                    pltpu.make_async_copy(tbl.at[idx_bufs[b]], row_bufs[b],
                                          sems[b]).wait()
                    fixup(idx_bufs[b], row_bufs[b])
                    pltpu.sync_copy(row_bufs[b],
                                    out.at[pl.ds(dst_base + j * CHUNK, CHUNK),
                                           :])

                    @pl.when(j + 2 < n_chunks)
                    def _():
                        pltpu.sync_copy(
                            ixs.at[pl.ds(src_base + (j + 2) * CHUNK, CHUNK)],
                            idx_bufs[b])
                        pltpu.async_copy(tbl.at[idx_bufs[b]], row_bufs[b],
                                         sems[b])
                return 0

            lax.fori_loop(0, n_chunks // 2, pair, 0)
            if n_chunks % 2:
                j = n_chunks - 1
                pltpu.make_async_copy(tbl.at[idx_bufs[0]], row_bufs[0],
                                      sems[0]).wait()
                fixup(idx_bufs[0], row_bufs[0])
                pltpu.sync_copy(row_bufs[0],
                                out.at[pl.ds(dst_base + j * CHUNK, CHUNK), :])

    return gather_kernel(idx_w, idx_b, idx_t, emb_word, emb_bigram, emb_trigram)


def _mlp_kernel(gw_ref, gb_ref, gt_ref, w1_ref, b1_ref, w2_ref, b2_ref,
                ee_ref, eo_ref, o_ref):
    xcat = jnp.concatenate(
        [gw_ref[...], gb_ref[...], gt_ref[...]], axis=1
    ).astype(jnp.bfloat16)
    h = jnp.dot(xcat, w1_ref[...], preferred_element_type=jnp.float32)
    h = jnp.maximum(h + b1_ref[...], 0.0).astype(jnp.bfloat16)
    o = jnp.dot(h, w2_ref[...], preferred_element_type=jnp.float32)
    o = o + b2_ref[...]
    ot = jnp.transpose(o, (1, 0))
    ev = jnp.dot(ot[0:NUM_CLASSES, :].astype(jnp.bfloat16), ee_ref[...],
                 preferred_element_type=jnp.float32)
    od = jnp.dot(ot[16:16 + NUM_CLASSES, :].astype(jnp.bfloat16), eo_ref[...],
                 preferred_element_type=jnp.float32)
    o_ref[...] = ev + od


def _tc_mlp(gw, gb, gt, fc1_w, fc1_b, fc2_w, fc2_b):
    nb = NT // BT
    w1 = fc1_w.T.astype(jnp.bfloat16)
    z = jnp.zeros((EMBED, HIDDEN), dtype=jnp.bfloat16)
    wd = []
    for s in range(3):
        ws = w1[s * EMBED:(s + 1) * EMBED]
        wd.append(jnp.block([[ws, z], [z, ws]]))
    wcat = jnp.concatenate(wd, axis=0)
    w2 = fc2_w.T.astype(jnp.bfloat16)
    w2p = jnp.zeros((2 * HIDDEN, 32), dtype=jnp.bfloat16)
    w2p = w2p.at[:HIDDEN, :NUM_CLASSES].set(w2)
    w2p = w2p.at[HIDDEN:, 16:16 + NUM_CLASSES].set(w2)
    b1d = jnp.concatenate([fc1_b, fc1_b]).reshape(1, 2 * HIDDEN)
    b2p = jnp.zeros((1, 32), dtype=jnp.float32)
    b2p = b2p.at[0, :NUM_CLASSES].set(fc2_b)
    b2p = b2p.at[0, 16:16 + NUM_CLASSES].set(fc2_b)

    pr = lax.iota(jnp.int32, BP).reshape(BP, 1)
    qc = lax.iota(jnp.int32, BT).reshape(1, BT)
    ee = (qc == 2 * pr).astype(jnp.bfloat16)
    eo = (qc == 2 * pr + 1).astype(jnp.bfloat16)

    g_spec = pl.BlockSpec((BP, 2 * EMBED), lambda i: (i, 0))
    return pl.pallas_call(
        _mlp_kernel,
        grid=(nb,),
        in_specs=[
            g_spec, g_spec, g_spec,
            pl.BlockSpec((3 * 2 * EMBED, 2 * HIDDEN), lambda i: (0, 0)),
            pl.BlockSpec((1, 2 * HIDDEN), lambda i: (0, 0)),
            pl.BlockSpec((2 * HIDDEN, 32), lambda i: (0, 0)),
            pl.BlockSpec((1, 32), lambda i: (0, 0)),
            pl.BlockSpec((BP, BT), lambda i: (0, 0)),
            pl.BlockSpec((BP, BT), lambda i: (0, 0)),
        ],
        out_specs=pl.BlockSpec((NUM_CLASSES, BT), lambda i: (0, i)),
        out_shape=jax.ShapeDtypeStruct((NUM_CLASSES, NT), jnp.float32),
    )(gw, gb, gt, wcat, b1d, w2p, b2p, ee, eo)


def kernel(x, emb_word, emb_bigram, emb_trigram, fc1_w, fc1_b, fc2_w, fc2_b):
    xt = jnp.transpose(x, (0, 2, 1))
    iw = xt[0].reshape(N)
    ib = xt[1].reshape(N)
    it = xt[2].reshape(N)
    outs = []
    for k in range(NSPLIT):
        gw, gb, gt = _sc_gather(iw, ib, it, emb_word, emb_bigram, emb_trigram,
                                k * NT)
        gw2 = gw.reshape(NT // 2, 2 * EMBED)
        gb2 = gb.reshape(NT // 2, 2 * EMBED)
        gt2 = gt.reshape(NT // 2, 2 * EMBED)
        outs.append(_tc_mlp(gw2, gb2, gt2, fc1_w, fc1_b, fc2_w, fc2_b))
    o_np = jnp.concatenate(outs, axis=1)
    return o_np.reshape(NUM_CLASSES, SEQ, BATCH).transpose(2, 1, 0)
